# baseline (speedup 1.0000x reference)
import functools

import jax
import jax.numpy as jnp
from jax import lax
from jax.experimental import pallas as pl
from jax.experimental.pallas import tpu as pltpu
from jax.experimental.pallas import tpu_sc as plsc

NC = 2
NS = 16
GROUP = 128
EDGE_ALIGN = 32768
NGRAPH = 128


def _round_up(n, m):
    return ((n + m - 1) // m) * m


@functools.lru_cache(maxsize=None)
def _segsum_kernel(n_src_pad, n_groups, n_dst_pad, w):
    full = (w == 64)
    if full:
        PT_G = n_groups // (NC * NS)
    else:
        PT_G = n_groups // NS
    acc_words = n_dst_pad * w
    zrow_words = GROUP * w
    per_gb_words = 2 * GROUP * w + 4 * GROUP
    gb = (2097151 - acc_words - NS * zrow_words) // (NS * per_gb_words)
    gb = max(1, min(gb, 16, PT_G))
    while PT_G % gb:
        gb -= 1
    NB = PT_G // gb
    R = n_dst_pad // NS
    nz_full, nz_tail = R // GROUP, R % GROUP
    n_sub = 1 if full else (64 // w) // NC
    mesh = plsc.VectorSubcoreMesh(
        core_axis_name="c", subcore_axis_name="s",
        num_cores=NC, num_subcores=NS)

    def body(table, srcg, dstg, out, s_idx0, s_idx1, d_idx0, d_idx1,
             rows0, rows1, zrow, acc, sg0, sg1, ss0, ss1):
        c = lax.axis_index("c")
        s = lax.axis_index("s")
        base = s * R
        g0 = (c * NS + s) * PT_G if full else s * PT_G
        s_idx = (s_idx0, s_idx1)
        d_idx = (d_idx0, d_idx1)
        rows = (rows0, rows1)
        sem_g = (sg0, sg1)
        sem_s = (ss0, ss1)
        zv = jnp.zeros((16,), jnp.float32)

        def zr(i, carry):
            for o in range(w // 16):
                zrow[i, pl.ds(16 * o, 16)] = zv
            return carry

        lax.fori_loop(0, GROUP, zr, 0)

        def load_idx(buf, b):
            gbase = g0 + b * gb
            pltpu.sync_copy(srcg.at[pl.ds(gbase, gb)], s_idx[buf])
            pltpu.sync_copy(dstg.at[pl.ds(gbase, gb)], d_idx[buf])

        for p in range(n_sub):
            tview = table if full else table.at[NC * p + c]
            oview = out.at[c] if full else out.at[NC * p + c]

            for i in range(nz_full):
                pltpu.sync_copy(zrow, acc.at[pl.ds(base + i * GROUP, GROUP)])
            if nz_tail:
                pltpu.sync_copy(
                    zrow.at[pl.ds(0, nz_tail)],
                    acc.at[pl.ds(base + nz_full * GROUP, nz_tail)])
            plsc.subcore_barrier()

            def fire_gathers(buf):
                for j in range(gb):
                    pltpu.async_copy(tview.at[s_idx[buf].at[j]],
                                     rows[buf].at[j], sem_g[buf])

            def process(buf, b):
                for j in range(gb):
                    pltpu.make_async_copy(
                        tview.at[s_idx[buf].at[j]], rows[buf].at[j],
                        sem_g[buf]).wait()
                sd = [pltpu.async_copy(rows[buf].at[j],
                                       acc.at[d_idx[buf].at[j]],
                                       sem_s[buf], add=True)
                      for j in range(gb)]
                for d in sd:
                    d.wait()

                @pl.when(b + 2 < NB)
                def _():
                    load_idx(buf, b + 2)
                    fire_gathers(buf)

            for buf in range(min(2, NB)):
                load_idx(buf, buf)
                fire_gathers(buf)

            def pair(k, carry):
                for buf in (0, 1):
                    process(buf, 2 * k + buf)
                return carry

            lax.fori_loop(0, NB // 2, pair, 0)
            if NB % 2:
                process((NB - 1) % 2, NB - 1)
            plsc.subcore_barrier()
            pltpu.sync_copy(acc.at[pl.ds(base, R)], oview.at[pl.ds(base, R)])
            plsc.subcore_barrier()

    n_split = NC if full else 64 // w
    return pl.kernel(
        body,
        out_type=jax.ShapeDtypeStruct((n_split, n_dst_pad, w), jnp.float32),
        mesh=mesh,
        scratch_types=[
            pltpu.VMEM((gb, GROUP), jnp.int32),
            pltpu.VMEM((gb, GROUP), jnp.int32),
            pltpu.VMEM((gb, GROUP), jnp.int32),
            pltpu.VMEM((gb, GROUP), jnp.int32),
            pltpu.VMEM((gb, GROUP, w), jnp.float32),
            pltpu.VMEM((gb, GROUP, w), jnp.float32),
            pltpu.VMEM((GROUP, w), jnp.float32),
            pltpu.VMEM_SHARED((n_dst_pad, w), jnp.float32),
            pltpu.SemaphoreType.DMA,
            pltpu.SemaphoreType.DMA,
            pltpu.SemaphoreType.DMA,
            pltpu.SemaphoreType.DMA,
        ],
        compiler_params=pltpu.CompilerParams(use_tc_tiling_on_sc=False),
    )


def _seg_full(table_flat, srcg, dstg, n_dst_pad):
    k = _segsum_kernel(table_flat.shape[0], srcg.shape[0], n_dst_pad, 64)
    return k(table_flat, srcg, dstg)


def _seg_q(table_q, srcg, dstg, n_dst_pad):
    k = _segsum_kernel(table_q.shape[1], srcg.shape[0], n_dst_pad, 16)
    return k(table_q, srcg, dstg)


def _pad_idx(src, dst, pad_src, pad_dst):
    n = src.shape[0]
    n_pad = _round_up(n, EDGE_ALIGN)
    pad = n_pad - n
    src = jnp.concatenate(
        [src.astype(jnp.int32), jnp.full((pad,), pad_src, jnp.int32)])
    dst = jnp.concatenate(
        [dst.astype(jnp.int32), jnp.full((pad,), pad_dst, jnp.int32)])
    return (src.reshape(n_pad // GROUP, GROUP),
            dst.reshape(n_pad // GROUP, GROUP))


def _embed_tc(idx_pad, table, n_pad, n_split):
    nblk = n_pad // 128
    wp = 64 // n_split
    vocab = table.shape[0]
    t_pad = jnp.zeros((128, 64), jnp.float32).at[:vocab].set(
        table.astype(jnp.float32))
    idx3 = idx_pad.reshape(nblk, 1, 128)

    def body(idx_ref, t_ref, out_ref):
        iv = idx_ref[0, 0, :]
        oh = (iv[:, None]
              == lax.broadcasted_iota(jnp.int32, (128, 128), 1))
        y = jnp.dot(oh.astype(jnp.float32), t_ref[...],
                    preferred_element_type=jnp.float32)
        if n_split == 1:
            out_ref[...] = y
        else:
            for q in range(n_split):
                out_ref[q] = y[:, q * wp:(q + 1) * wp]

    if n_split == 1:
        out_spec = pl.BlockSpec((128, 64), lambda i: (i, 0))
        out_sds = jax.ShapeDtypeStruct((n_pad, 64), jnp.float32)
    else:
        out_spec = pl.BlockSpec((n_split, 128, wp), lambda i: (0, i, 0))
        out_sds = jax.ShapeDtypeStruct((n_split, n_pad, wp), jnp.float32)
    return pl.pallas_call(
        body,
        grid=(nblk,),
        in_specs=[
            pl.BlockSpec((1, 1, 128), lambda i: (i, 0, 0)),
            pl.BlockSpec((128, 64), lambda i: (0, 0)),
        ],
        out_specs=out_spec,
        out_shape=out_sds,
    )(idx3, t_pad)


def _mm_e(ev_q, m_part, w, b, mode):
    n_pad = ev_q.shape[1]
    nblk = n_pad // 128
    w = w.astype(jnp.float32)
    b2d = b.reshape(1, -1).astype(jnp.float32)

    def body(a_ref, m_ref, w_ref, b_ref, *outs):
        xx = jnp.concatenate(
            [a_ref[0], a_ref[1], a_ref[2], a_ref[3],
             m_ref[0] + m_ref[1]], axis=-1)
        y = jnp.dot(xx, w_ref[...], preferred_element_type=jnp.float32)
        y = y + b_ref[...]
        for q in range(4):
            outs[0][q] = y[:, q * 16:(q + 1) * 16]
        if mode == "both":
            r = jnp.maximum(y, 0.0)
            for q in range(4):
                outs[1][q] = r[:, q * 16:(q + 1) * 16]

    a_spec = pl.BlockSpec((4, 128, 16), lambda i: (0, i, 0))
    m_spec = pl.BlockSpec((2, 128, 64), lambda i: (0, i, 0))
    out_sds = jax.ShapeDtypeStruct((4, n_pad, 16), jnp.float32)
    n_out = 2 if mode == "both" else 1
    return pl.pallas_call(
        body,
        grid=(nblk,),
        in_specs=[
            a_spec,
            m_spec,
            pl.BlockSpec((128, 64), lambda i: (0, 0)),
            pl.BlockSpec((1, 64), lambda i: (0, 0)),
        ],
        out_specs=[a_spec] * n_out,
        out_shape=[out_sds] * n_out,
    )(ev_q, m_part, w, b2d)


def _mm_v(xv_flat, m_q, w, b, mode):
    n_pad = xv_flat.shape[0]
    nblk = n_pad // 128
    w = w.astype(jnp.float32)
    b2d = b.reshape(1, -1).astype(jnp.float32)

    def body(a_ref, m_ref, w_ref, b_ref, out_ref):
        xx = jnp.concatenate(
            [a_ref[...], m_ref[0], m_ref[1], m_ref[2], m_ref[3]], axis=-1)
        y = jnp.dot(xx, w_ref[...], preferred_element_type=jnp.float32)
        y = y + b_ref[...]
        if mode == "relu":
            y = jnp.maximum(y, 0.0)
        out_ref[...] = y

    return pl.pallas_call(
        body,
        grid=(nblk,),
        in_specs=[
            pl.BlockSpec((128, 64), lambda i: (i, 0)),
            pl.BlockSpec((4, 128, 16), lambda i: (0, i, 0)),
            pl.BlockSpec((128, 64), lambda i: (0, 0)),
            pl.BlockSpec((1, 64), lambda i: (0, 0)),
        ],
        out_specs=pl.BlockSpec((128, 64), lambda i: (i, 0)),
        out_shape=jax.ShapeDtypeStruct((n_pad, 64), jnp.float32),
    )(xv_flat, m_q, w, b2d)


def _head(xg_part, eg_q, w1, b1, w2, b2):
    def body(xg_ref, eg_ref, w1_ref, b1_ref, w2_ref, b2_ref, out_ref):
        xx = jnp.concatenate(
            [xg_ref[0] + xg_ref[1],
             eg_ref[0], eg_ref[1], eg_ref[2], eg_ref[3]], axis=-1)
        h = jnp.dot(xx, w1_ref[...], preferred_element_type=jnp.float32)
        h = jnp.maximum(h + b1_ref[...], 0.0)
        out_ref[...] = (jnp.dot(h, w2_ref[...],
                                preferred_element_type=jnp.float32)
                        + b2_ref[...])

    return pl.pallas_call(
        body,
        grid=(1,),
        in_specs=[
            pl.BlockSpec((2, 128, 64), lambda i: (0, 0, 0)),
            pl.BlockSpec((4, 128, 16), lambda i: (0, 0, 0)),
            pl.BlockSpec((128, 128), lambda i: (0, 0)),
            pl.BlockSpec((1, 128), lambda i: (0, 0)),
            pl.BlockSpec((128, 1), lambda i: (0, 0)),
            pl.BlockSpec((1, 1), lambda i: (0, 0)),
        ],
        out_specs=pl.BlockSpec((128, 1), lambda i: (0, 0)),
        out_shape=jax.ShapeDtypeStruct((128, 1), jnp.float32),
    )(xg_part, eg_q, w1.astype(jnp.float32),
      b1.reshape(1, -1).astype(jnp.float32), w2.astype(jnp.float32),
      b2.reshape(1, -1).astype(jnp.float32))


def kernel(x, edge_attr, edge_index0, edge_index1, batch, e_batch,
           atom_table, hbond_table, W_e, b_e, W_v, b_v, W1, b1, W2, b2):
    nv = x.shape[0]
    nhe = edge_attr.shape[0]
    nvp = _round_up(nv + 1, 128)
    nhep = _round_up(nhe + 1, 128)
    ngp = _round_up(NGRAPH + 1, 128)

    x_pad = jnp.concatenate(
        [x.astype(jnp.int32), jnp.zeros((nvp - nv,), jnp.int32)])
    ea_pad = jnp.concatenate(
        [edge_attr.astype(jnp.int32), jnp.zeros((nhep - nhe,), jnp.int32)])

    iota_v = jnp.arange(nv, dtype=jnp.int32)
    iota_e = jnp.arange(nhe, dtype=jnp.int32)
    e0g, e1g = _pad_idx(edge_index0, edge_index1, nv, nhe)
    pvs, pvd = _pad_idx(iota_v, batch, 0, NGRAPH)
    pes, ped = _pad_idx(iota_e, e_batch, 0, NGRAPH)

    xv = _embed_tc(x_pad, atom_table, nvp, 1)
    ev = _embed_tc(ea_pad, hbond_table, nhep, 4)

    for layer in range(3):
        m_e = _seg_full(xv, e0g, e1g, nhep)
        if layer < 2:
            ev_raw, ev_next = _mm_e(ev, m_e, W_e, b_e, "both")
        else:
            (ev_raw,) = _mm_e(ev, m_e, W_e, b_e, "raw")
            ev_next = ev_raw
        m_v = _seg_q(ev_raw, e1g, e0g, nvp)
        xv = _mm_v(xv, m_v, W_v, b_v, "relu" if layer < 2 else "raw")
        ev = ev_next

    xg = _seg_full(xv, pvs, pvd, ngp)
    eg = _seg_q(ev, pes, ped, ngp)
    out = _head(xg, eg, W1, b1, W2, b2)
    return out.reshape(-1)


# speedup vs baseline: 1.1610x; 1.1610x over previous
import functools

import jax
import jax.numpy as jnp
from jax import lax
from jax.experimental import pallas as pl
from jax.experimental.pallas import tpu as pltpu
from jax.experimental.pallas import tpu_sc as plsc

NC = 2
NS = 16
GROUP = 128
EDGE_ALIGN = 32768
NGRAPH = 128


def _round_up(n, m):
    return ((n + m - 1) // m) * m


@functools.lru_cache(maxsize=None)
def _segsum_kernel(n_src_pad, n_groups, n_dst_pad, w):
    full = (w == 64)
    if full:
        PT_G = n_groups // (NC * NS)
    else:
        PT_G = n_groups // NS
    acc_words = n_dst_pad * w
    zrow_words = GROUP * w
    per_gb_words = 2 * GROUP * w + 4 * GROUP
    gb = (2097151 - acc_words - NS * zrow_words) // (NS * per_gb_words)
    gb = max(1, min(gb, 16, PT_G))
    while PT_G % gb:
        gb -= 1
    NB = PT_G // gb
    R = n_dst_pad // NS
    nz_full, nz_tail = R // GROUP, R % GROUP
    n_sub = 1 if full else (64 // w) // NC
    mesh = plsc.VectorSubcoreMesh(
        core_axis_name="c", subcore_axis_name="s",
        num_cores=NC, num_subcores=NS)

    def body(table, srcg, dstg, out, s_idx0, s_idx1, d_idx0, d_idx1,
             rows0, rows1, zrow, acc, sg0, sg1, ss0, ss1):
        c = lax.axis_index("c")
        s = lax.axis_index("s")
        base = s * R
        g0 = (c * NS + s) * PT_G if full else s * PT_G
        s_idx = (s_idx0, s_idx1)
        d_idx = (d_idx0, d_idx1)
        rows = (rows0, rows1)
        sem_g = (sg0, sg1)
        sem_s = (ss0, ss1)
        zv = jnp.zeros((16,), jnp.float32)

        def zr(i, carry):
            for o in range(w // 16):
                zrow[i, pl.ds(16 * o, 16)] = zv
            return carry

        lax.fori_loop(0, GROUP, zr, 0)

        def load_idx(buf, b):
            gbase = g0 + b * gb
            pltpu.sync_copy(srcg.at[pl.ds(gbase, gb)], s_idx[buf])
            pltpu.sync_copy(dstg.at[pl.ds(gbase, gb)], d_idx[buf])

        for p in range(n_sub):
            tview = table if full else table.at[NC * p + c]
            oview = out.at[c] if full else out.at[NC * p + c]

            for i in range(nz_full):
                pltpu.sync_copy(zrow, acc.at[pl.ds(base + i * GROUP, GROUP)])
            if nz_tail:
                pltpu.sync_copy(
                    zrow.at[pl.ds(0, nz_tail)],
                    acc.at[pl.ds(base + nz_full * GROUP, nz_tail)])
            plsc.subcore_barrier()

            def fire_gathers(buf):
                for j in range(gb):
                    pltpu.async_copy(tview.at[s_idx[buf].at[j]],
                                     rows[buf].at[j], sem_g[buf])

            def process(buf, b):
                for j in range(gb):
                    pltpu.make_async_copy(
                        tview.at[s_idx[buf].at[j]], rows[buf].at[j],
                        sem_g[buf]).wait()
                sd = [pltpu.async_copy(rows[buf].at[j],
                                       acc.at[d_idx[buf].at[j]],
                                       sem_s[buf], add=True)
                      for j in range(gb)]
                for d in sd:
                    d.wait()

                @pl.when(b + 2 < NB)
                def _():
                    load_idx(buf, b + 2)
                    fire_gathers(buf)

            for buf in range(min(2, NB)):
                load_idx(buf, buf)
                fire_gathers(buf)

            def pair(k, carry):
                for buf in (0, 1):
                    process(buf, 2 * k + buf)
                return carry

            lax.fori_loop(0, NB // 2, pair, 0)
            if NB % 2:
                process((NB - 1) % 2, NB - 1)
            plsc.subcore_barrier()
            pltpu.sync_copy(acc.at[pl.ds(base, R)], oview.at[pl.ds(base, R)])
            plsc.subcore_barrier()

    n_split = NC if full else 64 // w
    return pl.kernel(
        body,
        out_type=jax.ShapeDtypeStruct((n_split, n_dst_pad, w), jnp.float32),
        mesh=mesh,
        scratch_types=[
            pltpu.VMEM((gb, GROUP), jnp.int32),
            pltpu.VMEM((gb, GROUP), jnp.int32),
            pltpu.VMEM((gb, GROUP), jnp.int32),
            pltpu.VMEM((gb, GROUP), jnp.int32),
            pltpu.VMEM((gb, GROUP, w), jnp.float32),
            pltpu.VMEM((gb, GROUP, w), jnp.float32),
            pltpu.VMEM((GROUP, w), jnp.float32),
            pltpu.VMEM_SHARED((n_dst_pad, w), jnp.float32),
            pltpu.SemaphoreType.DMA,
            pltpu.SemaphoreType.DMA,
            pltpu.SemaphoreType.DMA,
            pltpu.SemaphoreType.DMA,
        ],
        compiler_params=pltpu.CompilerParams(use_tc_tiling_on_sc=False),
    )


def _seg_full(table_flat, srcg, dstg, n_dst_pad):
    k = _segsum_kernel(table_flat.shape[0], srcg.shape[0], n_dst_pad, 64)
    return k(table_flat, srcg, dstg)


def _seg_q(table_q, srcg, dstg, n_dst_pad):
    k = _segsum_kernel(table_q.shape[1], srcg.shape[0], n_dst_pad,
                       table_q.shape[2])
    return k(table_q, srcg, dstg)


def _pad_idx(src, dst, pad_src, pad_dst):
    n = src.shape[0]
    n_pad = _round_up(n, EDGE_ALIGN)
    pad = n_pad - n
    src = jnp.concatenate(
        [src.astype(jnp.int32), jnp.full((pad,), pad_src, jnp.int32)])
    dst = jnp.concatenate(
        [dst.astype(jnp.int32), jnp.full((pad,), pad_dst, jnp.int32)])
    return (src.reshape(n_pad // GROUP, GROUP),
            dst.reshape(n_pad // GROUP, GROUP))


def _embed_tc(idx_pad, table, n_pad, n_split):
    nblk = n_pad // 128
    wp = 64 // n_split
    vocab = table.shape[0]
    t_pad = jnp.zeros((128, 64), jnp.float32).at[:vocab].set(
        table.astype(jnp.float32))
    idx3 = idx_pad.reshape(nblk, 1, 128)

    def body(idx_ref, t_ref, out_ref):
        iv = idx_ref[0, 0, :]
        oh = (iv[:, None]
              == lax.broadcasted_iota(jnp.int32, (128, 128), 1))
        y = jnp.dot(oh.astype(jnp.float32), t_ref[...],
                    preferred_element_type=jnp.float32)
        if n_split == 1:
            out_ref[...] = y
        else:
            for q in range(n_split):
                out_ref[q] = y[:, q * wp:(q + 1) * wp]

    if n_split == 1:
        out_spec = pl.BlockSpec((128, 64), lambda i: (i, 0))
        out_sds = jax.ShapeDtypeStruct((n_pad, 64), jnp.float32)
    else:
        out_spec = pl.BlockSpec((n_split, 128, wp), lambda i: (0, i, 0))
        out_sds = jax.ShapeDtypeStruct((n_split, n_pad, wp), jnp.float32)
    return pl.pallas_call(
        body,
        grid=(nblk,),
        in_specs=[
            pl.BlockSpec((1, 1, 128), lambda i: (i, 0, 0)),
            pl.BlockSpec((128, 64), lambda i: (0, 0)),
        ],
        out_specs=out_spec,
        out_shape=out_sds,
    )(idx3, t_pad)


def _mm_e(ev_q, m_half, w, b, mode):
    n_pad = ev_q.shape[1]
    nblk = n_pad // 128
    w = w.astype(jnp.float32)
    b2d = b.reshape(1, -1).astype(jnp.float32)

    def body(a_ref, m_ref, w_ref, b_ref, *outs):
        xx = jnp.concatenate(
            [a_ref[0], a_ref[1], a_ref[2], a_ref[3],
             m_ref[0], m_ref[1]], axis=-1)
        y = jnp.dot(xx, w_ref[...], preferred_element_type=jnp.float32)
        y = y + b_ref[...]
        for q in range(4):
            outs[0][q] = y[:, q * 16:(q + 1) * 16]
        if mode == "both":
            r = jnp.maximum(y, 0.0)
            for q in range(4):
                outs[1][q] = r[:, q * 16:(q + 1) * 16]

    a_spec = pl.BlockSpec((4, 128, 16), lambda i: (0, i, 0))
    m_spec = pl.BlockSpec((2, 128, 32), lambda i: (0, i, 0))
    out_sds = jax.ShapeDtypeStruct((4, n_pad, 16), jnp.float32)
    n_out = 2 if mode == "both" else 1
    return pl.pallas_call(
        body,
        grid=(nblk,),
        in_specs=[
            a_spec,
            m_spec,
            pl.BlockSpec((128, 64), lambda i: (0, 0)),
            pl.BlockSpec((1, 64), lambda i: (0, 0)),
        ],
        out_specs=[a_spec] * n_out,
        out_shape=[out_sds] * n_out,
    )(ev_q, m_half, w, b2d)


def _mm_v(xv_h, m_q, w, b, mode):
    n_pad = xv_h.shape[1]
    nblk = n_pad // 128
    w = w.astype(jnp.float32)
    b2d = b.reshape(1, -1).astype(jnp.float32)

    def body(a_ref, m_ref, w_ref, b_ref, out_ref):
        xx = jnp.concatenate(
            [a_ref[0], a_ref[1],
             m_ref[0], m_ref[1], m_ref[2], m_ref[3]], axis=-1)
        y = jnp.dot(xx, w_ref[...], preferred_element_type=jnp.float32)
        y = y + b_ref[...]
        if mode == "relu":
            y = jnp.maximum(y, 0.0)
        out_ref[0] = y[:, :32]
        out_ref[1] = y[:, 32:]

    a_spec = pl.BlockSpec((2, 128, 32), lambda i: (0, i, 0))
    return pl.pallas_call(
        body,
        grid=(nblk,),
        in_specs=[
            a_spec,
            pl.BlockSpec((4, 128, 16), lambda i: (0, i, 0)),
            pl.BlockSpec((128, 64), lambda i: (0, 0)),
            pl.BlockSpec((1, 64), lambda i: (0, 0)),
        ],
        out_specs=a_spec,
        out_shape=jax.ShapeDtypeStruct((2, n_pad, 32), jnp.float32),
    )(xv_h, m_q, w, b2d)


def _head(xg_h, eg_q, w1, b1, w2, b2):
    def body(xg_ref, eg_ref, w1_ref, b1_ref, w2_ref, b2_ref, out_ref):
        xx = jnp.concatenate(
            [xg_ref[0], xg_ref[1],
             eg_ref[0], eg_ref[1], eg_ref[2], eg_ref[3]], axis=-1)
        h = jnp.dot(xx, w1_ref[...], preferred_element_type=jnp.float32)
        h = jnp.maximum(h + b1_ref[...], 0.0)
        out_ref[...] = (jnp.dot(h, w2_ref[...],
                                preferred_element_type=jnp.float32)
                        + b2_ref[...])

    return pl.pallas_call(
        body,
        grid=(1,),
        in_specs=[
            pl.BlockSpec((2, 128, 32), lambda i: (0, 0, 0)),
            pl.BlockSpec((4, 128, 16), lambda i: (0, 0, 0)),
            pl.BlockSpec((128, 128), lambda i: (0, 0)),
            pl.BlockSpec((1, 128), lambda i: (0, 0)),
            pl.BlockSpec((128, 1), lambda i: (0, 0)),
            pl.BlockSpec((1, 1), lambda i: (0, 0)),
        ],
        out_specs=pl.BlockSpec((128, 1), lambda i: (0, 0)),
        out_shape=jax.ShapeDtypeStruct((128, 1), jnp.float32),
    )(xg_h, eg_q, w1.astype(jnp.float32),
      b1.reshape(1, -1).astype(jnp.float32), w2.astype(jnp.float32),
      b2.reshape(1, -1).astype(jnp.float32))


def kernel(x, edge_attr, edge_index0, edge_index1, batch, e_batch,
           atom_table, hbond_table, W_e, b_e, W_v, b_v, W1, b1, W2, b2):
    nv = x.shape[0]
    nhe = edge_attr.shape[0]
    nvp = _round_up(nv + 1, 128)
    nhep = _round_up(nhe + 1, 128)
    ngp = _round_up(NGRAPH + 1, 128)

    x_pad = jnp.concatenate(
        [x.astype(jnp.int32), jnp.zeros((nvp - nv,), jnp.int32)])
    ea_pad = jnp.concatenate(
        [edge_attr.astype(jnp.int32), jnp.zeros((nhep - nhe,), jnp.int32)])

    iota_v = jnp.arange(nv, dtype=jnp.int32)
    iota_e = jnp.arange(nhe, dtype=jnp.int32)
    e0g, e1g = _pad_idx(edge_index0, edge_index1, nv, nhe)
    pvs, pvd = _pad_idx(iota_v, batch, 0, NGRAPH)
    pes, ped = _pad_idx(iota_e, e_batch, 0, NGRAPH)

    xv = _embed_tc(x_pad, atom_table, nvp, 2)
    ev = _embed_tc(ea_pad, hbond_table, nhep, 4)

    for layer in range(3):
        m_e = _seg_q(xv, e0g, e1g, nhep)
        if layer < 2:
            ev_raw, ev_next = _mm_e(ev, m_e, W_e, b_e, "both")
        else:
            (ev_raw,) = _mm_e(ev, m_e, W_e, b_e, "raw")
            ev_next = ev_raw
        m_v = _seg_q(ev_raw, e1g, e0g, nvp)
        xv = _mm_v(xv, m_v, W_v, b_v, "relu" if layer < 2 else "raw")
        ev = ev_next

    xg = _seg_q(xv, pvs, pvd, ngp)
    eg = _seg_q(ev, pes, ped, ngp)
    out = _head(xg, eg, W1, b1, W2, b2)
    return out.reshape(-1)


# speedup vs baseline: 1.8851x; 1.6237x over previous
import functools

import jax
import jax.numpy as jnp
from jax import lax
from jax.experimental import pallas as pl
from jax.experimental.pallas import tpu as pltpu
from jax.experimental.pallas import tpu_sc as plsc

NC = 2
NS = 16
GROUP = 128
W = 32
NGRAPH = 128
BLK = 512


def _round_up(n, m):
    return ((n + m - 1) // m) * m


@functools.lru_cache(maxsize=None)
def _segsum_kernel(n_src_pad, n_groups, n_dst_pad):
    PT_G = n_groups // NS
    acc_words = n_dst_pad * W
    zrow_words = 64 * W
    per_gb_words = 2 * GROUP * W + 4 * GROUP
    gb = (2097151 - acc_words - NS * zrow_words) // (NS * per_gb_words)
    gb = max(1, min(gb, 16, PT_G))
    while PT_G % gb or (PT_G // gb) % 2:
        gb -= 1
    NB = PT_G // gb
    R = n_dst_pad // NS
    nz_full, nz_tail = R // 64, R % 64
    mesh = plsc.VectorSubcoreMesh(
        core_axis_name="c", subcore_axis_name="s",
        num_cores=NC, num_subcores=NS)

    def body(table, srcg, dstg, out, s_idx0, s_idx1, d_idx0, d_idx1,
             rows0, rows1, zrow, acc, sg0, sg1, ss0, ss1):
        c = lax.axis_index("c")
        s = lax.axis_index("s")
        base = s * R
        g0 = s * PT_G
        s_idx = (s_idx0, s_idx1)
        d_idx = (d_idx0, d_idx1)
        rows = (rows0, rows1)
        sem_g = (sg0, sg1)
        sem_s = (ss0, ss1)
        tview = table.at[c]
        oview = out.at[c]
        zv = jnp.zeros((16,), jnp.float32)

        def zr(i, carry):
            for o in range(W // 16):
                zrow[i, pl.ds(16 * o, 16)] = zv
            return carry

        lax.fori_loop(0, 64, zr, 0)

        for i in range(nz_full):
            pltpu.sync_copy(zrow, acc.at[pl.ds(base + i * 64, 64)])
        if nz_tail:
            pltpu.sync_copy(zrow.at[pl.ds(0, nz_tail)],
                            acc.at[pl.ds(base + nz_full * 64, nz_tail)])
        plsc.subcore_barrier()

        def load_idx(buf, b):
            gbase = g0 + b * gb
            pltpu.sync_copy(srcg.at[pl.ds(gbase, gb)], s_idx[buf])
            pltpu.sync_copy(dstg.at[pl.ds(gbase, gb)], d_idx[buf])

        def fire_gathers(buf):
            for j in range(gb):
                pltpu.async_copy(tview.at[s_idx[buf].at[j]],
                                 rows[buf].at[j], sem_g[buf])

        def process(buf, b):
            for j in range(gb):
                pltpu.make_async_copy(
                    tview.at[s_idx[buf].at[j]], rows[buf].at[j],
                    sem_g[buf]).wait()
            sd = [pltpu.async_copy(rows[buf].at[j],
                                   acc.at[d_idx[buf].at[j]],
                                   sem_s[buf], add=True)
                  for j in range(gb)]
            for d in sd:
                d.wait()

            @pl.when(b + 2 < NB)
            def _():
                load_idx(buf, b + 2)
                fire_gathers(buf)

        for buf in range(min(2, NB)):
            load_idx(buf, buf)
            fire_gathers(buf)

        def pair(k, carry):
            for buf in (0, 1):
                process(buf, 2 * k + buf)
            return carry

        lax.fori_loop(0, NB // 2, pair, 0)
        if NB % 2:
            process((NB - 1) % 2, NB - 1)
        plsc.subcore_barrier()
        pltpu.sync_copy(acc.at[pl.ds(base, R)], oview.at[pl.ds(base, R)])

    return pl.kernel(
        body,
        out_type=jax.ShapeDtypeStruct((NC, n_dst_pad, W), jnp.float32),
        mesh=mesh,
        scratch_types=[
            pltpu.VMEM((gb, GROUP), jnp.int32),
            pltpu.VMEM((gb, GROUP), jnp.int32),
            pltpu.VMEM((gb, GROUP), jnp.int32),
            pltpu.VMEM((gb, GROUP), jnp.int32),
            pltpu.VMEM((gb, GROUP, W), jnp.float32),
            pltpu.VMEM((gb, GROUP, W), jnp.float32),
            pltpu.VMEM((64, W), jnp.float32),
            pltpu.VMEM_SHARED((n_dst_pad, W), jnp.float32),
            pltpu.SemaphoreType.DMA,
            pltpu.SemaphoreType.DMA,
            pltpu.SemaphoreType.DMA,
            pltpu.SemaphoreType.DMA,
        ],
        compiler_params=pltpu.CompilerParams(use_tc_tiling_on_sc=False),
    )


def _seg(table_pk, srcg, dstg, n_dst_pad):
    n_src = table_pk.shape[1] * 128 // W
    k = _segsum_kernel(n_src, srcg.shape[0], n_dst_pad)
    out = k(table_pk.reshape(NC, n_src, W), srcg, dstg)
    return out.reshape(NC, n_dst_pad * W // 128, 128)


def _pad_idx(src, dst, pad_src, pad_dst, align):
    n = src.shape[0]
    n_pad = _round_up(n, align)
    pad = n_pad - n
    src = jnp.concatenate(
        [src.astype(jnp.int32), jnp.full((pad,), pad_src, jnp.int32)])
    dst = jnp.concatenate(
        [dst.astype(jnp.int32), jnp.full((pad,), pad_dst, jnp.int32)])
    return (src.reshape(n_pad // GROUP, GROUP),
            dst.reshape(n_pad // GROUP, GROUP))


def _bd(w):
    w = w.astype(jnp.float32)
    w4 = w.reshape(4, 32, 2, 32).transpose(0, 2, 1, 3)
    eye = jnp.eye(4, dtype=jnp.float32)
    return jnp.einsum("ij,phab->phiajb", eye, w4).reshape(4, 2, 128, 128)


def _bd_tile(v):
    v = v.reshape(2, 32).astype(jnp.float32)
    return jnp.tile(v, (1, 4)).reshape(2, 1, 128)


def _embed_tc(idx_arr, table, n_pad):
    nblk = n_pad // BLK
    vocab = table.shape[0]
    t_pad = jnp.zeros((128, 64), jnp.float32).at[:vocab].set(
        table.astype(jnp.float32))

    def body(idx_ref, t_ref, out_ref):
        for h in range(2):
            pieces = []
            for k in range(4):
                iv = idx_ref[0, k, :]
                oh = (iv[:, None]
                      == lax.broadcasted_iota(jnp.int32, (128, 128), 1))
                pieces.append(jnp.dot(
                    oh.astype(jnp.float32), t_ref[:, h * 32:(h + 1) * 32],
                    preferred_element_type=jnp.float32))
            out_ref[h] = jnp.concatenate(pieces, axis=-1)

    return pl.pallas_call(
        body,
        grid=(nblk,),
        in_specs=[
            pl.BlockSpec((1, 4, 128), lambda i: (i, 0, 0)),
            pl.BlockSpec((128, 64), lambda i: (0, 0)),
        ],
        out_specs=pl.BlockSpec((2, 128, 128), lambda i: (0, i, 0)),
        out_shape=jax.ShapeDtypeStruct(
            (2, n_pad * W // 128, 128), jnp.float32),
    )(idx_arr, t_pad)


def _mm(a_pk, m_pk, w, b, mode):
    n_pk = a_pk.shape[1]
    nblk = n_pk // 128
    wbd = _bd(w)
    bbd = _bd_tile(b)

    def body(a_ref, m_ref, w_ref, b_ref, *outs):
        pieces = (a_ref[0], a_ref[1], m_ref[0], m_ref[1])
        for h in range(2):
            y = b_ref[h]
            for p in range(4):
                y = y + jnp.dot(pieces[p], w_ref[p, h],
                                preferred_element_type=jnp.float32)
            if mode == "relu":
                y = jnp.maximum(y, 0.0)
            outs[0][h] = y
            if mode == "both":
                outs[1][h] = jnp.maximum(y, 0.0)

    spec = pl.BlockSpec((2, 128, 128), lambda i: (0, i, 0))
    out_sds = jax.ShapeDtypeStruct((2, n_pk, 128), jnp.float32)
    n_out = 2 if mode == "both" else 1
    return pl.pallas_call(
        body,
        grid=(nblk,),
        in_specs=[
            spec,
            spec,
            pl.BlockSpec((4, 2, 128, 128), lambda i: (0, 0, 0, 0)),
            pl.BlockSpec((2, 1, 128), lambda i: (0, 0, 0)),
        ],
        out_specs=[spec] * n_out,
        out_shape=[out_sds] * n_out,
    )(a_pk, m_pk, wbd, bbd)


def _head(xg_pk, eg_pk, w1, b1, w2, b2):
    w1 = w1.astype(jnp.float32)
    w14 = w1.reshape(4, 32, 4, 32).transpose(0, 2, 1, 3)
    eye = jnp.eye(4, dtype=jnp.float32)
    w1bd = jnp.einsum("ij,phab->phiajb", eye, w14).reshape(4, 4, 128, 128)
    b1t = jnp.tile(b1.reshape(4, 32).astype(jnp.float32),
                   (1, 4)).reshape(4, 1, 128)
    w2p = w2.astype(jnp.float32).reshape(4, 32)
    w2bd = jnp.stack([
        jnp.einsum("ij,a->iaj", eye, w2p[j]).reshape(128, 4)
        for j in range(4)])
    b2v = jnp.full((1, 4), b2.reshape(()).astype(jnp.float32))

    def body(xg_ref, eg_ref, w1_ref, b1_ref, w2_ref, b2_ref, out_ref):
        pieces = (xg_ref[0], xg_ref[1], eg_ref[0], eg_ref[1])
        out = b2_ref[...]
        for j in range(4):
            h = b1_ref[j]
            for p in range(4):
                h = h + jnp.dot(pieces[p], w1_ref[p, j],
                                preferred_element_type=jnp.float32)
            h = jnp.maximum(h, 0.0)
            out = out + jnp.dot(h, w2_ref[j],
                                preferred_element_type=jnp.float32)
        out_ref[...] = out

    return pl.pallas_call(
        body,
        grid=(1,),
        in_specs=[
            pl.BlockSpec((2, 32, 128), lambda i: (0, 0, 0)),
            pl.BlockSpec((2, 32, 128), lambda i: (0, 0, 0)),
            pl.BlockSpec((4, 4, 128, 128), lambda i: (0, 0, 0, 0)),
            pl.BlockSpec((4, 1, 128), lambda i: (0, 0, 0)),
            pl.BlockSpec((4, 128, 4), lambda i: (0, 0, 0)),
            pl.BlockSpec((1, 4), lambda i: (0, 0)),
        ],
        out_specs=pl.BlockSpec((32, 4), lambda i: (0, 0)),
        out_shape=jax.ShapeDtypeStruct((32, 4), jnp.float32),
    )(xg_pk, eg_pk, w1bd, b1t, w2bd, b2v)


def kernel(x, edge_attr, edge_index0, edge_index1, batch, e_batch,
           atom_table, hbond_table, W_e, b_e, W_v, b_v, W1, b1, W2, b2):
    nv = x.shape[0]
    nhe = edge_attr.shape[0]
    nvp = _round_up(nv + 1, BLK)
    nhep = _round_up(nhe + 1, BLK)
    ngp = _round_up(NGRAPH + 1, 128)

    def arrange_idx(ids, n_pad):
        pad = jnp.zeros((n_pad - ids.shape[0],), jnp.int32)
        flat = jnp.concatenate([ids.astype(jnp.int32), pad])
        return flat.reshape(n_pad // BLK, 128, 4).transpose(0, 2, 1)

    x_arr = arrange_idx(x, nvp)
    ea_arr = arrange_idx(edge_attr, nhep)

    iota_v = jnp.arange(nv, dtype=jnp.int32)
    iota_e = jnp.arange(nhe, dtype=jnp.int32)
    e0me, e1me = _pad_idx(edge_index0, edge_index1, nv, nhe, 40960)
    e1mv, e0mv = _pad_idx(edge_index1, edge_index0, nhe, nv, 12288)
    pvs, pvd = _pad_idx(iota_v, batch, 0, NGRAPH, 32768)
    pes, ped = _pad_idx(iota_e, e_batch, 0, NGRAPH, 32768)

    xv = _embed_tc(x_arr, atom_table, nvp)
    ev = _embed_tc(ea_arr, hbond_table, nhep)

    for layer in range(3):
        m_e = _seg(xv, e0me, e1me, nhep)
        if layer < 2:
            ev_raw, ev_next = _mm(ev, m_e, W_e, b_e, "both")
        else:
            (ev_raw,) = _mm(ev, m_e, W_e, b_e, "raw")
            ev_next = ev_raw
        m_v = _seg(ev_raw, e1mv, e0mv, nvp)
        (xv,) = _mm(xv, m_v, W_v, b_v, "relu" if layer < 2 else "raw")
        ev = ev_next

    xg = _seg(xv, pvs, pvd, ngp)
    eg = _seg(ev, pes, ped, ngp)
    out = _head(xg, eg, W1, b1, W2, b2)
    return out.reshape(-1)


# speedup vs baseline: 1.8921x; 1.0037x over previous
import functools

import jax
import jax.numpy as jnp
from jax import lax
from jax.experimental import pallas as pl
from jax.experimental.pallas import tpu as pltpu
from jax.experimental.pallas import tpu_sc as plsc

NC = 2
NS = 16
GROUP = 128
W = 32
NGRAPH = 128
BLK = 512


def _round_up(n, m):
    return ((n + m - 1) // m) * m


@functools.lru_cache(maxsize=None)
def _segsum_kernel(n_src_pad, n_groups, n_dst_pad):
    PT_G = n_groups // NS
    acc_words = n_dst_pad * W
    zrow_words = 64 * W
    per_gb_words = 2 * GROUP * W + 4 * GROUP
    gb = (2097151 - acc_words - NS * zrow_words) // (NS * per_gb_words)
    gb = max(1, min(gb, 16, PT_G))
    while PT_G % gb or (PT_G // gb) % 2:
        gb -= 1
    NB = PT_G // gb
    R = n_dst_pad // NS
    nz_full, nz_tail = R // 64, R % 64
    mesh = plsc.VectorSubcoreMesh(
        core_axis_name="c", subcore_axis_name="s",
        num_cores=NC, num_subcores=NS)

    def body(table, srcg, dstg, out, s_idx0, s_idx1, d_idx0, d_idx1,
             rows0, rows1, zrow, acc, sg0, sg1, ss0, ss1, sz):
        c = lax.axis_index("c")
        s = lax.axis_index("s")
        base = s * R
        g0 = s * PT_G
        s_idx = (s_idx0, s_idx1)
        d_idx = (d_idx0, d_idx1)
        rows = (rows0, rows1)
        sem_g = (sg0, sg1)
        sem_s = (ss0, ss1)
        tview = table.at[c]
        oview = out.at[c]
        zv = jnp.zeros((16,), jnp.float32)

        def zr(i, carry):
            for o in range(W // 16):
                zrow[i, pl.ds(16 * o, 16)] = zv
            return carry

        lax.fori_loop(0, 64, zr, 0)

        zd = [pltpu.async_copy(zrow, acc.at[pl.ds(base + i * 64, 64)], sz)
              for i in range(nz_full)]
        if nz_tail:
            zd.append(pltpu.async_copy(
                zrow.at[pl.ds(0, nz_tail)],
                acc.at[pl.ds(base + nz_full * 64, nz_tail)], sz))
        for d in zd:
            d.wait()
        plsc.subcore_barrier()

        def load_idx(buf, b):
            gbase = g0 + b * gb
            pltpu.sync_copy(srcg.at[pl.ds(gbase, gb)], s_idx[buf])
            pltpu.sync_copy(dstg.at[pl.ds(gbase, gb)], d_idx[buf])

        def fire_gathers(buf):
            for j in range(gb):
                pltpu.async_copy(tview.at[s_idx[buf].at[j]],
                                 rows[buf].at[j], sem_g[buf])

        def process(buf, b):
            for j in range(gb):
                pltpu.make_async_copy(
                    tview.at[s_idx[buf].at[j]], rows[buf].at[j],
                    sem_g[buf]).wait()
            sd = [pltpu.async_copy(rows[buf].at[j],
                                   acc.at[d_idx[buf].at[j]],
                                   sem_s[buf], add=True)
                  for j in range(gb)]
            for d in sd:
                d.wait()

            @pl.when(b + 2 < NB)
            def _():
                load_idx(buf, b + 2)
                fire_gathers(buf)

        for buf in range(min(2, NB)):
            load_idx(buf, buf)
            fire_gathers(buf)

        def pair(k, carry):
            for buf in (0, 1):
                process(buf, 2 * k + buf)
            return carry

        lax.fori_loop(0, NB // 2, pair, 0)
        if NB % 2:
            process((NB - 1) % 2, NB - 1)
        plsc.subcore_barrier()
        pltpu.sync_copy(acc.at[pl.ds(base, R)], oview.at[pl.ds(base, R)])

    return pl.kernel(
        body,
        out_type=jax.ShapeDtypeStruct((NC, n_dst_pad, W), jnp.float32),
        mesh=mesh,
        scratch_types=[
            pltpu.VMEM((gb, GROUP), jnp.int32),
            pltpu.VMEM((gb, GROUP), jnp.int32),
            pltpu.VMEM((gb, GROUP), jnp.int32),
            pltpu.VMEM((gb, GROUP), jnp.int32),
            pltpu.VMEM((gb, GROUP, W), jnp.float32),
            pltpu.VMEM((gb, GROUP, W), jnp.float32),
            pltpu.VMEM((64, W), jnp.float32),
            pltpu.VMEM_SHARED((n_dst_pad, W), jnp.float32),
            pltpu.SemaphoreType.DMA,
            pltpu.SemaphoreType.DMA,
            pltpu.SemaphoreType.DMA,
            pltpu.SemaphoreType.DMA,
            pltpu.SemaphoreType.DMA,
        ],
        compiler_params=pltpu.CompilerParams(use_tc_tiling_on_sc=False),
    )


def _seg(table_pk, srcg, dstg, n_dst_pad):
    n_src = table_pk.shape[1] * 128 // W
    k = _segsum_kernel(n_src, srcg.shape[0], n_dst_pad)
    out = k(table_pk.reshape(NC, n_src, W), srcg, dstg)
    return out.reshape(NC, n_dst_pad * W // 128, 128)


def _pad_idx(src, dst, pad_src, pad_dst, align):
    n = src.shape[0]
    n_pad = _round_up(n, align)
    pad = n_pad - n
    src = jnp.concatenate(
        [src.astype(jnp.int32), jnp.full((pad,), pad_src, jnp.int32)])
    dst = jnp.concatenate(
        [dst.astype(jnp.int32), jnp.full((pad,), pad_dst, jnp.int32)])
    return (src.reshape(n_pad // GROUP, GROUP),
            dst.reshape(n_pad // GROUP, GROUP))


def _bd(w):
    w = w.astype(jnp.float32)
    w4 = w.reshape(4, 32, 2, 32).transpose(0, 2, 1, 3)
    eye = jnp.eye(4, dtype=jnp.float32)
    return jnp.einsum("ij,phab->phiajb", eye, w4).reshape(4, 2, 128, 128)


def _bd_tile(v):
    v = v.reshape(2, 32).astype(jnp.float32)
    return jnp.tile(v, (1, 4)).reshape(2, 1, 128)


def _embed_tc(idx_arr, table, n_pad):
    nblk = n_pad // BLK
    vocab = table.shape[0]
    t_pad = jnp.zeros((128, 64), jnp.float32).at[:vocab].set(
        table.astype(jnp.float32))

    def body(idx_ref, t_ref, out_ref):
        for h in range(2):
            pieces = []
            for k in range(4):
                iv = idx_ref[0, k, :]
                oh = (iv[:, None]
                      == lax.broadcasted_iota(jnp.int32, (128, 128), 1))
                pieces.append(jnp.dot(
                    oh.astype(jnp.float32), t_ref[:, h * 32:(h + 1) * 32],
                    preferred_element_type=jnp.float32))
            out_ref[h] = jnp.concatenate(pieces, axis=-1)

    return pl.pallas_call(
        body,
        grid=(nblk,),
        in_specs=[
            pl.BlockSpec((1, 4, 128), lambda i: (i, 0, 0)),
            pl.BlockSpec((128, 64), lambda i: (0, 0)),
        ],
        out_specs=pl.BlockSpec((2, 128, 128), lambda i: (0, i, 0)),
        out_shape=jax.ShapeDtypeStruct(
            (2, n_pad * W // 128, 128), jnp.float32),
    )(idx_arr, t_pad)


def _mm(a_pk, m_pk, w, b, mode):
    n_pk = a_pk.shape[1]
    nblk = n_pk // 128
    wbd = _bd(w)
    bbd = _bd_tile(b)

    def body(a_ref, m_ref, w_ref, b_ref, *outs):
        pieces = (a_ref[0], a_ref[1], m_ref[0], m_ref[1])
        for h in range(2):
            y = b_ref[h]
            for p in range(4):
                y = y + jnp.dot(pieces[p], w_ref[p, h],
                                preferred_element_type=jnp.float32)
            if mode == "relu":
                y = jnp.maximum(y, 0.0)
            outs[0][h] = y
            if mode == "both":
                outs[1][h] = jnp.maximum(y, 0.0)

    spec = pl.BlockSpec((2, 128, 128), lambda i: (0, i, 0))
    out_sds = jax.ShapeDtypeStruct((2, n_pk, 128), jnp.float32)
    n_out = 2 if mode == "both" else 1
    return pl.pallas_call(
        body,
        grid=(nblk,),
        in_specs=[
            spec,
            spec,
            pl.BlockSpec((4, 2, 128, 128), lambda i: (0, 0, 0, 0)),
            pl.BlockSpec((2, 1, 128), lambda i: (0, 0, 0)),
        ],
        out_specs=[spec] * n_out,
        out_shape=[out_sds] * n_out,
    )(a_pk, m_pk, wbd, bbd)


def _head(xg_pk, eg_pk, w1, b1, w2, b2):
    w1 = w1.astype(jnp.float32)
    w14 = w1.reshape(4, 32, 4, 32).transpose(0, 2, 1, 3)
    eye = jnp.eye(4, dtype=jnp.float32)
    w1bd = jnp.einsum("ij,phab->phiajb", eye, w14).reshape(4, 4, 128, 128)
    b1t = jnp.tile(b1.reshape(4, 32).astype(jnp.float32),
                   (1, 4)).reshape(4, 1, 128)
    w2p = w2.astype(jnp.float32).reshape(4, 32)
    w2bd = jnp.stack([
        jnp.einsum("ij,a->iaj", eye, w2p[j]).reshape(128, 4)
        for j in range(4)])
    b2v = jnp.full((1, 4), b2.reshape(()).astype(jnp.float32))

    def body(xg_ref, eg_ref, w1_ref, b1_ref, w2_ref, b2_ref, out_ref):
        pieces = (xg_ref[0], xg_ref[1], eg_ref[0], eg_ref[1])
        out = b2_ref[...]
        for j in range(4):
            h = b1_ref[j]
            for p in range(4):
                h = h + jnp.dot(pieces[p], w1_ref[p, j],
                                preferred_element_type=jnp.float32)
            h = jnp.maximum(h, 0.0)
            out = out + jnp.dot(h, w2_ref[j],
                                preferred_element_type=jnp.float32)
        out_ref[...] = out

    return pl.pallas_call(
        body,
        grid=(1,),
        in_specs=[
            pl.BlockSpec((2, 32, 128), lambda i: (0, 0, 0)),
            pl.BlockSpec((2, 32, 128), lambda i: (0, 0, 0)),
            pl.BlockSpec((4, 4, 128, 128), lambda i: (0, 0, 0, 0)),
            pl.BlockSpec((4, 1, 128), lambda i: (0, 0, 0)),
            pl.BlockSpec((4, 128, 4), lambda i: (0, 0, 0)),
            pl.BlockSpec((1, 4), lambda i: (0, 0)),
        ],
        out_specs=pl.BlockSpec((32, 4), lambda i: (0, 0)),
        out_shape=jax.ShapeDtypeStruct((32, 4), jnp.float32),
    )(xg_pk, eg_pk, w1bd, b1t, w2bd, b2v)


def kernel(x, edge_attr, edge_index0, edge_index1, batch, e_batch,
           atom_table, hbond_table, W_e, b_e, W_v, b_v, W1, b1, W2, b2):
    nv = x.shape[0]
    nhe = edge_attr.shape[0]
    nvp = _round_up(nv + 1, BLK)
    nhep = _round_up(nhe + 1, BLK)
    ngp = _round_up(NGRAPH + 1, 128)

    def arrange_idx(ids, n_pad):
        pad = jnp.zeros((n_pad - ids.shape[0],), jnp.int32)
        flat = jnp.concatenate([ids.astype(jnp.int32), pad])
        return flat.reshape(n_pad // BLK, 128, 4).transpose(0, 2, 1)

    x_arr = arrange_idx(x, nvp)
    ea_arr = arrange_idx(edge_attr, nhep)

    iota_v = jnp.arange(nv, dtype=jnp.int32)
    iota_e = jnp.arange(nhe, dtype=jnp.int32)
    e0me, e1me = _pad_idx(edge_index0, edge_index1, nv, nhe, 40960)
    e1mv, e0mv = _pad_idx(edge_index1, edge_index0, nhe, nv, 12288)
    pvs, pvd = _pad_idx(iota_v, batch, 0, NGRAPH, 32768)
    pes, ped = _pad_idx(iota_e, e_batch, 0, NGRAPH, 32768)

    xv = _embed_tc(x_arr, atom_table, nvp)
    ev = _embed_tc(ea_arr, hbond_table, nhep)

    for layer in range(3):
        m_e = _seg(xv, e0me, e1me, nhep)
        if layer < 2:
            ev_raw, ev_next = _mm(ev, m_e, W_e, b_e, "both")
        else:
            (ev_raw,) = _mm(ev, m_e, W_e, b_e, "raw")
            ev_next = ev_raw
        m_v = _seg(ev_raw, e1mv, e0mv, nvp)
        (xv,) = _mm(xv, m_v, W_v, b_v, "relu" if layer < 2 else "raw")
        ev = ev_next

    xg = _seg(xv, pvs, pvd, ngp)
    eg = _seg(ev, pes, ped, ngp)
    out = _head(xg, eg, W1, b1, W2, b2)
    return out.reshape(-1)
